# Initial kernel scaffold; baseline (speedup 1.0000x reference)
import jax
import jax.numpy as jnp
from jax.experimental import pallas as pl


def kernel(x, emb, W_ih, W_hh, b_ih, b_hh, W_cls, b_cls):
    raise NotImplementedError("write your pallas kernel here")


# speedup vs baseline: 1.8821x; 1.8821x over previous
import functools

import jax
import jax.numpy as jnp
from jax import lax
from jax.experimental import pallas as pl
from jax.experimental.pallas import tpu as pltpu
from jax.experimental.pallas import tpu_sc as plsc


def _gather_sc(idx_flat, emb):
    N = idx_flat.shape[0]
    E = emb.shape[1]
    info = plsc.get_sparse_core_info()
    nw = info.num_cores * info.num_subcores
    per_w = N // nw
    ch = 80
    nch = per_w // ch
    assert per_w % ch == 0 and N % nw == 0

    mesh = plsc.VectorSubcoreMesh(core_axis_name="c", subcore_axis_name="s")

    @functools.partial(
        pl.kernel,
        mesh=mesh,
        out_type=jax.ShapeDtypeStruct((N, E), jnp.float32),
        scratch_types=[
            pltpu.VMEM((ch,), jnp.int32),
            pltpu.VMEM((ch, E), jnp.float32),
            pltpu.SemaphoreType.DMA,
        ],
    )
    def gk(idx_hbm, emb_hbm, out_hbm, idx_v, rows_v, sem):
        wid = lax.axis_index("s") * info.num_cores + lax.axis_index("c")
        base = wid * per_w

        def chunk(j, carry):
            off = base + j * ch
            pltpu.sync_copy(idx_hbm.at[pl.ds(off, ch)], idx_v)
            pltpu.async_copy(emb_hbm.at[idx_v], rows_v, sem).wait()
            pltpu.sync_copy(rows_v, out_hbm.at[pl.ds(off, ch)])
            return carry

        lax.fori_loop(0, nch, chunk, 0)

    return gk(idx_flat, emb)


def _lstm_body(e_ref, wx_ref, wh_ref, b_ref, wc_ref, bc_ref, out_ref,
               h_scr, c_scr):
    T = e_ref.shape[0]
    H = h_scr.shape[1]
    h_scr[...] = jnp.zeros_like(h_scr)
    c_scr[...] = jnp.zeros_like(c_scr)
    wx = wx_ref[...]
    wh = wh_ref[...]
    b = b_ref[...]

    def step(t, carry):
        et = e_ref[t]
        h = h_scr[...]
        gates = jnp.dot(et, wx, preferred_element_type=jnp.float32)
        gates = gates + jnp.dot(h, wh, preferred_element_type=jnp.float32)
        gates = gates + b
        ig = jax.nn.sigmoid(gates[:, :H])
        fg = jax.nn.sigmoid(gates[:, H:2 * H])
        gg = jnp.tanh(gates[:, 2 * H:3 * H])
        og = jax.nn.sigmoid(gates[:, 3 * H:])
        c = fg * c_scr[...] + ig * gg
        h_scr[...] = og * jnp.tanh(c)
        c_scr[...] = c
        return carry

    lax.fori_loop(0, T, step, 0)
    out_ref[...] = (
        jnp.dot(h_scr[...], wc_ref[...], preferred_element_type=jnp.float32)
        + bc_ref[...]
    )


def _lstm_tc(e_tm, wx, wh, bias, wc, bc, tb=256):
    T, B, E = e_tm.shape
    H = wh.shape[0]
    nb = B // tb
    return pl.pallas_call(
        _lstm_body,
        grid=(nb,),
        in_specs=[
            pl.BlockSpec((T, tb, E), lambda i: (0, i, 0)),
            pl.BlockSpec((E, 4 * H), lambda i: (0, 0)),
            pl.BlockSpec((H, 4 * H), lambda i: (0, 0)),
            pl.BlockSpec((1, 4 * H), lambda i: (0, 0)),
            pl.BlockSpec((H, 128), lambda i: (0, 0)),
            pl.BlockSpec((1, 128), lambda i: (0, 0)),
        ],
        out_specs=pl.BlockSpec((tb, 128), lambda i: (i, 0)),
        out_shape=jax.ShapeDtypeStruct((B, 128), jnp.float32),
        scratch_shapes=[
            pltpu.VMEM((tb, H), jnp.float32),
            pltpu.VMEM((tb, H), jnp.float32),
        ],
    )(e_tm, wx, wh, bias, wc, bc)


def kernel(x, emb, W_ih, W_hh, b_ih, b_hh, W_cls, b_cls):
    B, T = x.shape
    E = emb.shape[1]
    H = W_hh.shape[1]
    ncls = W_cls.shape[0]

    idx_tm = x.T.reshape(-1).astype(jnp.int32)
    e_flat = _gather_sc(idx_tm, emb.astype(jnp.float32))
    e_tm = e_flat.reshape(T, B, E)

    wx = W_ih.T
    wh = W_hh.T
    bias = (b_ih + b_hh).reshape(1, 4 * H)
    wc = jnp.zeros((H, 128), jnp.float32).at[:, :ncls].set(W_cls.T)
    bc = jnp.zeros((1, 128), jnp.float32).at[:, :ncls].set(b_cls)

    out = _lstm_tc(e_tm, wx, wh, bias, wc, bc)
    return out[:, :ncls]


# speedup vs baseline: 1.9797x; 1.0519x over previous
import functools

import jax
import jax.numpy as jnp
from jax import lax
from jax.experimental import pallas as pl
from jax.experimental.pallas import tpu as pltpu
from jax.experimental.pallas import tpu_sc as plsc


def _gather_sc(idx_flat, emb):
    N = idx_flat.shape[0]
    E = emb.shape[1]
    info = plsc.get_sparse_core_info()
    nw = info.num_cores * info.num_subcores
    per_w = N // nw
    ch = 80
    nch = per_w // ch
    assert per_w % ch == 0 and N % nw == 0

    mesh = plsc.VectorSubcoreMesh(core_axis_name="c", subcore_axis_name="s")

    @functools.partial(
        pl.kernel,
        mesh=mesh,
        out_type=jax.ShapeDtypeStruct((N, E), jnp.float32),
        scratch_types=[
            pltpu.VMEM((ch,), jnp.int32),
            pltpu.VMEM((ch, E), jnp.float32),
            pltpu.SemaphoreType.DMA,
        ],
    )
    def gk(idx_hbm, emb_hbm, out_hbm, idx_v, rows_v, sem):
        wid = lax.axis_index("s") * info.num_cores + lax.axis_index("c")
        base = wid * per_w

        def chunk(j, carry):
            off = base + j * ch
            pltpu.sync_copy(idx_hbm.at[pl.ds(off, ch)], idx_v)
            pltpu.async_copy(emb_hbm.at[idx_v], rows_v, sem).wait()
            pltpu.sync_copy(rows_v, out_hbm.at[pl.ds(off, ch)])
            return carry

        lax.fori_loop(0, nch, chunk, 0)

    return gk(idx_flat, emb)


def _lstm_body(e_ref, wx_ref, wh_ref, b_ref, wc_ref, bc_ref, out_ref,
               h_scr, c_scr):
    T = e_ref.shape[0]
    H = h_scr.shape[1]
    h_scr[...] = jnp.zeros_like(h_scr)
    c_scr[...] = jnp.zeros_like(c_scr)
    wx = wx_ref[...]
    wh = wh_ref[...]
    b = b_ref[...]

    def step(t, carry):
        et = e_ref[t]
        h = h_scr[...]
        gates = jnp.dot(et, wx, preferred_element_type=jnp.float32)
        gates = gates + jnp.dot(h, wh, preferred_element_type=jnp.float32)
        gates = gates + b
        ig = jax.nn.sigmoid(gates[:, :H])
        fg = jax.nn.sigmoid(gates[:, H:2 * H])
        gg = jnp.tanh(gates[:, 2 * H:3 * H])
        og = jax.nn.sigmoid(gates[:, 3 * H:])
        c = fg * c_scr[...] + ig * gg
        h_scr[...] = og * jnp.tanh(c)
        c_scr[...] = c
        return carry

    lax.fori_loop(0, T, step, 0)
    out_ref[...] = (
        jnp.dot(h_scr[...], wc_ref[...], preferred_element_type=jnp.float32)
        + bc_ref[...]
    )


def _lstm_tc(e_tm, wx, wh, bias, wc, bc, tb=512):
    T, B, E = e_tm.shape
    H = wh.shape[0]
    nb = B // tb
    return pl.pallas_call(
        _lstm_body,
        grid=(nb,),
        in_specs=[
            pl.BlockSpec((T, tb, E), lambda i: (0, i, 0)),
            pl.BlockSpec((E, 4 * H), lambda i: (0, 0)),
            pl.BlockSpec((H, 4 * H), lambda i: (0, 0)),
            pl.BlockSpec((1, 4 * H), lambda i: (0, 0)),
            pl.BlockSpec((H, 128), lambda i: (0, 0)),
            pl.BlockSpec((1, 128), lambda i: (0, 0)),
        ],
        out_specs=pl.BlockSpec((tb, 128), lambda i: (i, 0)),
        out_shape=jax.ShapeDtypeStruct((B, 128), jnp.float32),
        scratch_shapes=[
            pltpu.VMEM((tb, H), jnp.float32),
            pltpu.VMEM((tb, H), jnp.float32),
        ],
    )(e_tm, wx, wh, bias, wc, bc)


def kernel(x, emb, W_ih, W_hh, b_ih, b_hh, W_cls, b_cls):
    B, T = x.shape
    E = emb.shape[1]
    H = W_hh.shape[1]
    ncls = W_cls.shape[0]

    idx_tm = x.T.reshape(-1).astype(jnp.int32)
    e_flat = _gather_sc(idx_tm, emb.astype(jnp.float32))
    e_tm = e_flat.reshape(T, B, E)

    wx = W_ih.T
    wh = W_hh.T
    bias = (b_ih + b_hh).reshape(1, 4 * H)
    wc = jnp.zeros((H, 128), jnp.float32).at[:, :ncls].set(W_cls.T)
    bc = jnp.zeros((1, 128), jnp.float32).at[:, :ncls].set(b_cls)

    out = _lstm_tc(e_tm, wx, wh, bias, wc, bc)
    return out[:, :ncls]


# speedup vs baseline: 2.0736x; 1.0474x over previous
import functools

import jax
import jax.numpy as jnp
from jax import lax
from jax.experimental import pallas as pl
from jax.experimental.pallas import tpu as pltpu
from jax.experimental.pallas import tpu_sc as plsc


def _gather_sc(idx_flat, emb):
    N = idx_flat.shape[0]
    E = emb.shape[1]
    info = plsc.get_sparse_core_info()
    nw = info.num_cores * info.num_subcores
    per_w = N // nw
    ch = 80
    nch = per_w // ch
    assert per_w % ch == 0 and N % nw == 0

    mesh = plsc.VectorSubcoreMesh(core_axis_name="c", subcore_axis_name="s")

    @functools.partial(
        pl.kernel,
        mesh=mesh,
        out_type=jax.ShapeDtypeStruct((N, E), jnp.float32),
        scratch_types=[
            pltpu.VMEM((ch,), jnp.int32),
            pltpu.VMEM((ch, E), jnp.float32),
            pltpu.SemaphoreType.DMA,
        ],
    )
    def gk(idx_hbm, emb_hbm, out_hbm, idx_v, rows_v, sem):
        wid = lax.axis_index("s") * info.num_cores + lax.axis_index("c")
        base = wid * per_w

        def chunk(j, carry):
            off = base + j * ch
            pltpu.sync_copy(idx_hbm.at[pl.ds(off, ch)], idx_v)
            pltpu.async_copy(emb_hbm.at[idx_v], rows_v, sem).wait()
            pltpu.sync_copy(rows_v, out_hbm.at[pl.ds(off, ch)])
            return carry

        lax.fori_loop(0, nch, chunk, 0)

    return gk(idx_flat, emb)


def _lstm_body(e_ref, wx_ref, wh_ref, b_ref, wc_ref, bc_ref, out_ref,
               h_scr, c_scr):
    T = e_ref.shape[0]
    H = h_scr.shape[1]
    h_scr[...] = jnp.zeros_like(h_scr)
    c_scr[...] = jnp.zeros_like(c_scr)
    wx = wx_ref[...]
    wh = wh_ref[...]
    b = b_ref[...]

    def step(t, carry):
        et = e_ref[t]
        h = h_scr[...]
        gates = jnp.dot(et, wx, preferred_element_type=jnp.float32)
        gates = gates + jnp.dot(h, wh, preferred_element_type=jnp.float32)
        gates = gates + b
        ig = jax.nn.sigmoid(gates[:, :H])
        fg = jax.nn.sigmoid(gates[:, H:2 * H])
        gg = jnp.tanh(gates[:, 2 * H:3 * H])
        og = jax.nn.sigmoid(gates[:, 3 * H:])
        c = fg * c_scr[...] + ig * gg
        h_scr[...] = og * jnp.tanh(c)
        c_scr[...] = c
        return carry

    lax.fori_loop(0, T, step, 0)
    out_ref[...] = (
        jnp.dot(h_scr[...], wc_ref[...], preferred_element_type=jnp.float32)
        + bc_ref[...]
    )


def _lstm_tc(e_tm, wx, wh, bias, wc, bc, tb=1024):
    T, B, E = e_tm.shape
    H = wh.shape[0]
    nb = B // tb
    return pl.pallas_call(
        _lstm_body,
        grid=(nb,),
        in_specs=[
            pl.BlockSpec((T, tb, E), lambda i: (0, i, 0)),
            pl.BlockSpec((E, 4 * H), lambda i: (0, 0)),
            pl.BlockSpec((H, 4 * H), lambda i: (0, 0)),
            pl.BlockSpec((1, 4 * H), lambda i: (0, 0)),
            pl.BlockSpec((H, 128), lambda i: (0, 0)),
            pl.BlockSpec((1, 128), lambda i: (0, 0)),
        ],
        out_specs=pl.BlockSpec((tb, 128), lambda i: (i, 0)),
        out_shape=jax.ShapeDtypeStruct((B, 128), jnp.float32),
        scratch_shapes=[
            pltpu.VMEM((tb, H), jnp.float32),
            pltpu.VMEM((tb, H), jnp.float32),
        ],
    )(e_tm, wx, wh, bias, wc, bc)


def kernel(x, emb, W_ih, W_hh, b_ih, b_hh, W_cls, b_cls):
    B, T = x.shape
    E = emb.shape[1]
    H = W_hh.shape[1]
    ncls = W_cls.shape[0]

    idx_tm = x.T.reshape(-1).astype(jnp.int32)
    e_flat = _gather_sc(idx_tm, emb.astype(jnp.float32))
    e_tm = e_flat.reshape(T, B, E)

    wx = W_ih.T
    wh = W_hh.T
    bias = (b_ih + b_hh).reshape(1, 4 * H)
    wc = jnp.zeros((H, 128), jnp.float32).at[:, :ncls].set(W_cls.T)
    bc = jnp.zeros((1, 128), jnp.float32).at[:, :ncls].set(b_cls)

    out = _lstm_tc(e_tm, wx, wh, bias, wc, bc)
    return out[:, :ncls]


# speedup vs baseline: 2.3365x; 1.1268x over previous
import functools

import jax
import jax.numpy as jnp
from jax import lax
from jax.experimental import pallas as pl
from jax.experimental.pallas import tpu as pltpu
from jax.experimental.pallas import tpu_sc as plsc


def _gather_sc(idx_flat, emb):
    N = idx_flat.shape[0]
    E = emb.shape[1]
    info = plsc.get_sparse_core_info()
    nw = info.num_cores * info.num_subcores
    per_w = N // nw
    ch = 80
    nch = per_w // ch
    assert per_w % ch == 0 and N % nw == 0

    mesh = plsc.VectorSubcoreMesh(core_axis_name="c", subcore_axis_name="s")

    @functools.partial(
        pl.kernel,
        mesh=mesh,
        out_type=jax.ShapeDtypeStruct((N, E), jnp.float32),
        scratch_types=[
            pltpu.VMEM((ch,), jnp.int32),
            pltpu.VMEM((ch, E), jnp.float32),
            pltpu.SemaphoreType.DMA,
        ],
    )
    def gk(idx_hbm, emb_hbm, out_hbm, idx_v, rows_v, sem):
        wid = lax.axis_index("s") * info.num_cores + lax.axis_index("c")
        base = wid * per_w

        def chunk(j, carry):
            off = base + j * ch
            pltpu.sync_copy(idx_hbm.at[pl.ds(off, ch)], idx_v)
            pltpu.async_copy(emb_hbm.at[idx_v], rows_v, sem).wait()
            pltpu.sync_copy(rows_v, out_hbm.at[pl.ds(off, ch)])
            return carry

        lax.fori_loop(0, nch, chunk, 0)

    return gk(idx_flat, emb)


def _sig(x):
    return 0.5 * jnp.tanh(0.5 * x) + 0.5


def _lstm_body(e_ref, wcat_ref, b_ref, wc_ref, bc_ref, out_ref,
               xh_scr, c_scr):
    T = e_ref.shape[0]
    E = e_ref.shape[2]
    H = c_scr.shape[1]
    xh_scr[...] = jnp.zeros_like(xh_scr)
    c_scr[...] = jnp.zeros_like(c_scr)

    def step(t, carry):
        xh_scr[:, :E] = e_ref[t]
        gates = jnp.dot(xh_scr[...], wcat_ref[...],
                        preferred_element_type=jnp.float32)
        b = b_ref[...]
        ig = _sig(gates[:, :H] + b[:, :H])
        fg = _sig(gates[:, H:2 * H] + b[:, H:2 * H])
        gg = jnp.tanh(gates[:, 2 * H:3 * H] + b[:, 2 * H:3 * H])
        og = _sig(gates[:, 3 * H:] + b[:, 3 * H:])
        c = fg * c_scr[...] + ig * gg
        c_scr[...] = c
        xh_scr[:, E:] = og * jnp.tanh(c)
        return carry

    lax.fori_loop(0, T, step, 0)
    out_ref[...] = (
        jnp.dot(xh_scr[:, E:], wc_ref[...], preferred_element_type=jnp.float32)
        + bc_ref[...]
    )


def _lstm_tc(e_tm, wcat, bias, wc, bc, tb=1024):
    T, B, E = e_tm.shape
    H = wc.shape[0]
    nb = B // tb
    return pl.pallas_call(
        _lstm_body,
        grid=(nb,),
        in_specs=[
            pl.BlockSpec((T, tb, E), lambda i: (0, i, 0)),
            pl.BlockSpec((E + H, 4 * H), lambda i: (0, 0)),
            pl.BlockSpec((1, 4 * H), lambda i: (0, 0)),
            pl.BlockSpec((H, 128), lambda i: (0, 0)),
            pl.BlockSpec((1, 128), lambda i: (0, 0)),
        ],
        out_specs=pl.BlockSpec((tb, 128), lambda i: (i, 0)),
        out_shape=jax.ShapeDtypeStruct((B, 128), jnp.float32),
        scratch_shapes=[
            pltpu.VMEM((tb, E + H), jnp.float32),
            pltpu.VMEM((tb, H), jnp.float32),
        ],
    )(e_tm, wcat, bias, wc, bc)


def kernel(x, emb, W_ih, W_hh, b_ih, b_hh, W_cls, b_cls):
    B, T = x.shape
    E = emb.shape[1]
    H = W_hh.shape[1]
    ncls = W_cls.shape[0]

    idx_tm = x.T.reshape(-1).astype(jnp.int32)
    e_flat = _gather_sc(idx_tm, emb.astype(jnp.float32))
    e_tm = e_flat.reshape(T, B, E)

    wcat = jnp.concatenate([W_ih.T, W_hh.T], axis=0)
    bias = (b_ih + b_hh).reshape(1, 4 * H)
    wc = jnp.zeros((H, 128), jnp.float32).at[:, :ncls].set(W_cls.T)
    bc = jnp.zeros((1, 128), jnp.float32).at[:, :ncls].set(b_cls)

    out = _lstm_tc(e_tm, wcat, bias, wc, bc)
    return out[:, :ncls]
